# baseline (speedup 1.0000x reference)
import functools

import jax
import jax.numpy as jnp
from jax import lax
from jax.experimental import pallas as pl
from jax.experimental.pallas import tpu as pltpu
from jax.experimental.pallas import tpu_sc as plsc

K = 16
T = 8
FB = 32


def _box_kernel(x_ref, win_ref):
    x = x_ref[0]
    Tt = x.shape[0]
    z = jnp.zeros((T,) + x.shape[1:], x.dtype)
    xp = jnp.concatenate([z, x, z], axis=0)
    s2 = xp[:-1] + xp[1:]
    s4 = s2[:-2] + s2[2:]
    s8 = s4[:-4] + s4[4:]
    s16 = s8[:-8] + s8[8:]
    win = s16[:Tt] + xp[2 * T :]
    r = win.astype(jnp.bfloat16).astype(jnp.float32)
    bits = jax.lax.bitcast_convert_type(r, jnp.int32)
    v2 = bits.reshape(Tt // 2, 2, *bits.shape[1:])
    packed = (v2[:, 1] & jnp.int32(-65536)) | jax.lax.shift_right_logical(
        v2[:, 0], 16
    )
    win_ref[...] = packed[None]


def _box_filter(x_in, nb):
    B, Tt, F, C = x_in.shape
    return pl.pallas_call(
        _box_kernel,
        grid=(nb, F // FB),
        in_specs=[pl.BlockSpec((1, Tt, FB, C), lambda b, f: (b, 0, f, 0))],
        out_specs=pl.BlockSpec((1, Tt // 2, FB, C), lambda b, f: (b, 0, f, 0)),
        out_shape=jax.ShapeDtypeStruct((nb, Tt // 2, F, C), jnp.int32),
    )(x_in)


CB = 32


def _mix_matrix(F):
    import numpy as np
    series = np.arange(1, K + 1)
    omega = np.arange(F)
    idx = np.clip(omega[:, None] * series[None, :], 0, F - 1)
    M = np.zeros((F, F), dtype=np.float32)
    np.add.at(M, (np.repeat(omega, K), idx.reshape(-1)), 1.0)
    return M


TB = 32


def _tc_mix_kernel(xprev_ref, xcur_ref, xnext_ref, m_ref, out_ref, *, nt):
    tc = pl.program_id(1)
    xe = jnp.concatenate(
        [xprev_ref[0], xcur_ref[0], xnext_ref[0]], axis=0
    )
    tglob = (tc * TB - TB) + jax.lax.broadcasted_iota(jnp.int32, (3 * TB, 1, 1), 0)
    valid = (tglob >= 0) & (tglob < nt * TB)
    xe = jnp.where(valid, xe, 0.0)
    s2 = xe[:-1] + xe[1:]
    s4 = s2[:-2] + s2[2:]
    s8 = s4[:-4] + s4[4:]
    s16 = s8[:-8] + s8[8:]
    win = s16[TB - T : 2 * TB - T] + xe[TB + T : 2 * TB + T]
    F_ = win.shape[1]
    C_ = win.shape[2]
    w2 = win.reshape(TB // 2, 2 * F_, C_)
    mblk = m_ref[...]
    outs = [
        jax.lax.dot(mblk, w2[i], preferred_element_type=jnp.float32)
        for i in range(TB // 2)
    ]
    out_ref[...] = jnp.stack(outs).reshape(1, TB, F_, C_)


def _mix_tc(x_in, nb_sc):
    import numpy as np
    B, Tt, F, C = x_in.shape
    nb = B - nb_sc
    nt = Tt // TB
    M = _mix_matrix(F)
    mblk = np.zeros((2 * F, 2 * F), dtype=np.float32)
    mblk[:F, :F] = M
    mblk[F:, F:] = M
    mblk = jnp.asarray(mblk)
    xspec = lambda fn: pl.BlockSpec((1, TB, F, C), fn)
    return pl.pallas_call(
        functools.partial(_tc_mix_kernel, nt=nt),
        grid=(nb, nt),
        in_specs=[
            xspec(lambda b, t: (b + nb_sc, jnp.maximum(t - 1, 0), 0, 0)),
            xspec(lambda b, t: (b + nb_sc, t, 0, 0)),
            xspec(lambda b, t: (b + nb_sc, jnp.minimum(t + 1, nt - 1), 0, 0)),
            pl.BlockSpec((2 * F, 2 * F), lambda b, t: (0, 0)),
        ],
        out_specs=pl.BlockSpec((1, TB, F, C), lambda b, t: (b, t, 0, 0)),
        out_shape=jax.ShapeDtypeStruct((nb, Tt, F, C), jnp.float32),
    )(x_in, x_in, x_in, mblk)


def _kmax_groups(F):
    groups = []
    for o in range(F):
        km = K if o == 0 else min(K, (F - 1) // o)
        if groups and groups[-1][2] == km:
            groups[-1] = (groups[-1][0], o + 1, km)
        else:
            groups.append((o, o + 1, km))
    return groups


def _harmonic_mix_sc(win, Tt):
    B, TP, F, C = win.shape
    NLANES = 16
    NCH = C // NLANES
    info = plsc.get_sparse_core_info()
    nworkers = info.num_cores * info.num_subcores
    npairs = B * TP
    ppw = npairs // nworkers
    groups = _kmax_groups(F)
    mesh = plsc.VectorSubcoreMesh(core_axis_name="c", subcore_axis_name="s")

    @functools.partial(
        pl.kernel,
        mesh=mesh,
        out_type=jax.ShapeDtypeStruct((B, Tt, F, C), jnp.float32),
        compiler_params=pltpu.CompilerParams(needs_layout_passes=False),
        scratch_types=[
            pltpu.VMEM((2, F, C), jnp.int32),
            pltpu.VMEM((2, 2, F, C), jnp.float32),
            pltpu.SemaphoreType.DMA((2,)),
            pltpu.SemaphoreType.DMA((2,)),
        ],
    )
    def mix(win_hbm, out_hbm, wslab, oslab, sem_in, sem_out):
        wid = lax.axis_index("s") * info.num_cores + lax.axis_index("c")
        base = wid * ppw

        def load_row(p, f):
            out = []
            for ch in range(NCH):
                v = wslab[p, f, pl.ds(ch * NLANES, NLANES)]
                out.append(
                    (
                        plsc.bitcast(v << 16, jnp.float32),
                        plsc.bitcast(v & jnp.int32(-65536), jnp.float32),
                    )
                )
            return out

        def in_copy(i):
            s = base + i
            p = i % 2
            return pltpu.make_async_copy(
                win_hbm.at[s // TP, s % TP], wslab.at[p], sem_in.at[p]
            )

        def out_copy(i):
            s = base + i
            p = i % 2
            return pltpu.make_async_copy(
                oslab.at[p], out_hbm.at[s // TP, pl.ds(2 * (s % TP), 2)],
                sem_out.at[p],
            )

        in_copy(0).start()

        def pair_body(i, carry):
            @pl.when(i + 1 < ppw)
            def _():
                in_copy(i + 1).start()

            in_copy(i).wait()

            @pl.when(i >= 2)
            def _():
                out_copy(i - 2).wait()

            p = i % 2
            w127 = load_row(p, F - 1)
            for (lo, hi, km) in groups:
                cclip = float(K - km)

                def o_body(o, carry2, km=km, cclip=cclip):
                    if km < K:
                        accs = [(wl * cclip, wh * cclip) for (wl, wh) in w127]
                        k0 = 1
                    else:
                        accs = load_row(p, o)
                        k0 = 2
                    for k in range(k0, km + 1):
                        row = load_row(p, k * o)
                        accs = [
                            (al + rl, ah + rh)
                            for (al, ah), (rl, rh) in zip(accs, row)
                        ]
                    for ch in range(NCH):
                        oslab[p, 0, o, pl.ds(ch * NLANES, NLANES)] = accs[ch][0]
                        oslab[p, 1, o, pl.ds(ch * NLANES, NLANES)] = accs[ch][1]
                    return carry2

                L = hi - lo
                if km < K and L >= 4:

                    def o2_body(j, carry2, lo=lo, ob=o_body):
                        o = lo + 2 * j
                        ob(o, 0)
                        ob(o + 1, 0)
                        return carry2

                    lax.fori_loop(0, L // 2, o2_body, 0)
                    if L % 2:
                        o_body(hi - 1, 0)
                else:
                    lax.fori_loop(lo, hi, o_body, 0)
            out_copy(i).start()
            return carry

        lax.fori_loop(0, ppw, pair_body, 0)
        out_copy(ppw - 2).wait()
        out_copy(ppw - 1).wait()

    return mix(win)


NB_SC = 2


def kernel(x_in):
    win = _box_filter(x_in, NB_SC)
    sc_out = _harmonic_mix_sc(win, x_in.shape[1])
    tc_out = _mix_tc(x_in, NB_SC)
    return jnp.concatenate([sc_out, tc_out], axis=0)


# speedup vs baseline: 1.0145x; 1.0145x over previous
import functools

import jax
import jax.numpy as jnp
from jax import lax
from jax.experimental import pallas as pl
from jax.experimental.pallas import tpu as pltpu
from jax.experimental.pallas import tpu_sc as plsc

K = 16
T = 8
FB = 32


def _box_kernel(x_ref, win_ref):
    x = x_ref[0]
    Tt = x.shape[0]
    z = jnp.zeros((T,) + x.shape[1:], x.dtype)
    xp = jnp.concatenate([z, x, z], axis=0)
    s2 = xp[:-1] + xp[1:]
    s4 = s2[:-2] + s2[2:]
    s8 = s4[:-4] + s4[4:]
    s16 = s8[:-8] + s8[8:]
    win = s16[:Tt] + xp[2 * T :]
    r = win.astype(jnp.bfloat16).astype(jnp.float32)
    bits = jax.lax.bitcast_convert_type(r, jnp.int32)
    v2 = bits.reshape(Tt // 2, 2, *bits.shape[1:])
    packed = (v2[:, 1] & jnp.int32(-65536)) | jax.lax.shift_right_logical(
        v2[:, 0], 16
    )
    win_ref[...] = packed[None]


def _box_filter(x_in):
    B, Tt, F, C = x_in.shape
    return pl.pallas_call(
        _box_kernel,
        grid=(B, F // FB),
        in_specs=[pl.BlockSpec((1, Tt, FB, C), lambda b, f: (b, 0, f, 0))],
        out_specs=pl.BlockSpec((1, Tt // 2, FB, C), lambda b, f: (b, 0, f, 0)),
        out_shape=jax.ShapeDtypeStruct((B, Tt // 2, F, C), jnp.int32),
    )(x_in)


def _kmax_groups(F):
    groups = []
    for o in range(F):
        km = K if o == 0 else min(K, (F - 1) // o)
        if groups and groups[-1][2] == km:
            groups[-1] = (groups[-1][0], o + 1, km)
        else:
            groups.append((o, o + 1, km))
    return groups


def _harmonic_mix_sc(win, Tt):
    B, TP, F, C = win.shape
    NLANES = 16
    NCH = C // NLANES
    info = plsc.get_sparse_core_info()
    nworkers = info.num_cores * info.num_subcores
    npairs = B * TP
    ppw = npairs // nworkers
    groups = _kmax_groups(F)
    mesh = plsc.VectorSubcoreMesh(core_axis_name="c", subcore_axis_name="s")

    @functools.partial(
        pl.kernel,
        mesh=mesh,
        out_type=jax.ShapeDtypeStruct((B, Tt, F, C), jnp.float32),
        compiler_params=pltpu.CompilerParams(needs_layout_passes=False),
        scratch_types=[
            pltpu.VMEM((2, F, C), jnp.int32),
            pltpu.VMEM((2, 2, F, C), jnp.float32),
            pltpu.SemaphoreType.DMA((2,)),
            pltpu.SemaphoreType.DMA((2,)),
        ],
    )
    def mix(win_hbm, out_hbm, wslab, oslab, sem_in, sem_out):
        wid = lax.axis_index("s") * info.num_cores + lax.axis_index("c")
        base = wid * ppw

        def load_row(p, f):
            out = []
            for ch in range(NCH):
                v = wslab[p, f, pl.ds(ch * NLANES, NLANES)]
                out.append(
                    (
                        plsc.bitcast(v << 16, jnp.float32),
                        plsc.bitcast(v & jnp.int32(-65536), jnp.float32),
                    )
                )
            return out

        def in_copy(i):
            s = base + i
            p = i % 2
            return pltpu.make_async_copy(
                win_hbm.at[s // TP, s % TP], wslab.at[p], sem_in.at[p]
            )

        def out_copy(i):
            s = base + i
            p = i % 2
            return pltpu.make_async_copy(
                oslab.at[p], out_hbm.at[s // TP, pl.ds(2 * (s % TP), 2)],
                sem_out.at[p],
            )

        in_copy(0).start()

        def pair_body(i, carry):
            @pl.when(i + 1 < ppw)
            def _():
                in_copy(i + 1).start()

            in_copy(i).wait()

            @pl.when(i >= 2)
            def _():
                out_copy(i - 2).wait()

            p = i % 2
            w127 = load_row(p, F - 1)
            for (lo, hi, km) in groups:
                cclip = float(K - km)

                def o_body(o, carry2, km=km, cclip=cclip):
                    if km < K:
                        accs = [(wl * cclip, wh * cclip) for (wl, wh) in w127]
                        k0 = 1
                    else:
                        accs = load_row(p, o)
                        k0 = 2
                    for k in range(k0, km + 1):
                        row = load_row(p, k * o)
                        accs = [
                            (al + rl, ah + rh)
                            for (al, ah), (rl, rh) in zip(accs, row)
                        ]
                    for ch in range(NCH):
                        oslab[p, 0, o, pl.ds(ch * NLANES, NLANES)] = accs[ch][0]
                        oslab[p, 1, o, pl.ds(ch * NLANES, NLANES)] = accs[ch][1]
                    return carry2

                L = hi - lo
                if km < K and L >= 4:

                    def o2_body(j, carry2, lo=lo, ob=o_body):
                        o = lo + 2 * j
                        ob(o, 0)
                        ob(o + 1, 0)
                        return carry2

                    lax.fori_loop(0, L // 2, o2_body, 0)
                    if L % 2:
                        o_body(hi - 1, 0)
                else:
                    lax.fori_loop(lo, hi, o_body, 0)
            out_copy(i).start()
            return carry

        lax.fori_loop(0, ppw, pair_body, 0)
        out_copy(ppw - 2).wait()
        out_copy(ppw - 1).wait()

    return mix(win)


def kernel(x_in):
    win = _box_filter(x_in)
    return _harmonic_mix_sc(win, x_in.shape[1])
